# Initial kernel scaffold; baseline (speedup 1.0000x reference)
import jax
import jax.numpy as jnp
from jax.experimental import pallas as pl


def kernel(x, edge_index, batch, W1, a_src1, a_dst1, b1, W2, a_src2, a_dst2, b2, Wc, bc):
    raise NotImplementedError("write your pallas kernel here")


# speedup vs baseline: 1.0780x; 1.0780x over previous
import functools

import jax
import jax.numpy as jnp
from jax.experimental import pallas as pl
from jax.experimental.pallas import tpu as pltpu

N_NODES = 10000
N_EDGES = 320000
IN_CH = 128
HID = 256
HEADS = 8
NUM_CLASSES = 10
NUM_GRAPHS = 64
C1 = HID // HEADS


def _matmul_kernel(x_ref, w_ref, o_ref):
    o_ref[...] = jnp.dot(x_ref[...], w_ref[...],
                         preferred_element_type=jnp.float32)


def _matmul(x, w, block_rows=1000):
    m, k = x.shape
    k2, n = w.shape
    assert k == k2 and m % block_rows == 0
    grid = (m // block_rows,)
    return pl.pallas_call(
        _matmul_kernel,
        grid=grid,
        in_specs=[
            pl.BlockSpec((block_rows, k), lambda i: (i, 0)),
            pl.BlockSpec((k, n), lambda i: (0, 0)),
        ],
        out_specs=pl.BlockSpec((block_rows, n), lambda i: (i, 0)),
        out_shape=jax.ShapeDtypeStruct((m, n), jnp.float32),
    )(x, w)


def _gat_layer(h_in, src, dst, W, a_src, a_dst, heads, ch):
    n = h_in.shape[0]
    h = _matmul(h_in, W).reshape(n, heads, ch)
    alpha_src = (h * a_src[None]).sum(-1)
    alpha_dst = (h * a_dst[None]).sum(-1)
    alpha = alpha_src[src] + alpha_dst[dst]
    alpha = jax.nn.leaky_relu(alpha, 0.2)
    w = jnp.exp(alpha)
    den = jax.ops.segment_sum(w, dst, num_segments=n)
    num = jax.ops.segment_sum(h[src] * w[..., None], dst, num_segments=n)
    return num, den


def kernel(x, edge_index, batch, W1, a_src1, a_dst1, b1, W2, a_src2, a_dst2, b2, Wc, bc):
    n = x.shape[0]
    loop = jnp.arange(n, dtype=edge_index.dtype)
    src = jnp.concatenate([edge_index[0], loop])
    dst = jnp.concatenate([edge_index[1], loop])

    num1, den1 = _gat_layer(x, src, dst, W1, a_src1, a_dst1, HEADS, C1)
    h1 = (num1 / den1[..., None]).reshape(n, HEADS * C1) + b1
    h1 = jax.nn.relu(h1)

    num2, den2 = _gat_layer(h1, src, dst, W2, a_src2, a_dst2, HEADS, HID)
    h2 = (num2 / den2[..., None]).mean(axis=1) + b2

    sums = jax.ops.segment_sum(h2, batch, num_segments=NUM_GRAPHS)
    cnt = jax.ops.segment_sum(jnp.ones((n, 1), h2.dtype), batch,
                              num_segments=NUM_GRAPHS)
    pooled = sums / jnp.maximum(cnt, 1.0)
    return pooled @ Wc + bc


# speedup vs baseline: 9.7078x; 9.0057x over previous
import numpy as np

import jax
import jax.numpy as jnp
from jax import lax
from jax.experimental import pallas as pl
from jax.experimental.pallas import tpu as pltpu
from jax.experimental.pallas import tpu_sc as plsc

NN = 10000
NE = 320000
NE3 = 330240
NPAD = NE3 - (NE + NN)
NACC = 10112
IN_CH = 128
HID = 256
HEADS = 8
C1 = 32
NUM_CLASSES = 10
NUM_GRAPHS = 64
NC = 2
NS = 16
K = 32
NCH1 = NE // NS // K
NCH3 = NE3 // NS // K

_MESH = dict(core_axis_name="c", subcore_axis_name="s", num_cores=NC,
             num_subcores=NS)

def _iota16():
    return lax.iota(jnp.int32, 16)


def _bcast(vec, h):
    idx = _iota16() * 0 + h
    return vec.at[idx].get(mode="promise_in_bounds")


def _gat(vec, idx):
    return vec.at[idx].get(mode="promise_in_bounds")


def _lrelu(x):
    return jnp.where(x >= 0, x, 0.2 * x)


def _stage_a_body(x_ref, w1_ref, bs_ref, bd_ref, h1s_ref, a1s_ref, init_ref):
    h1h = jnp.dot(x_ref[...], w1_ref[...], preferred_element_type=jnp.float32)
    ps = jnp.dot(h1h, bs_ref[0], preferred_element_type=jnp.float32)
    pd = jnp.dot(h1h, bd_ref[0], preferred_element_type=jnp.float32)
    h1s_ref[...] = h1h
    zero8 = jnp.zeros((h1h.shape[0], 8), jnp.float32)
    a1s_ref[...] = jnp.concatenate([ps, pd, zero8], axis=1)
    wself = jnp.exp(_lrelu(ps + pd))
    parts = [h1h[:, 32 * h:32 * (h + 1)] * wself[:, h:h + 1] for h in range(4)]
    zero12 = jnp.zeros((h1h.shape[0], 12), jnp.float32)
    init_ref[...] = jnp.concatenate(parts + [wself, zero12], axis=1)


def _stage_a(x, W1, Bsrc1, Bdst1):
    B = 1000
    grid = (NN // B, 2)
    return pl.pallas_call(
        _stage_a_body,
        grid=grid,
        in_specs=[
            pl.BlockSpec((B, IN_CH), lambda i, s: (i, 0)),
            pl.BlockSpec((IN_CH, 128), lambda i, s: (0, s)),
            pl.BlockSpec((1, 128, 4), lambda i, s: (s, 0, 0)),
            pl.BlockSpec((1, 128, 4), lambda i, s: (s, 0, 0)),
        ],
        out_specs=[
            pl.BlockSpec((B, 128), lambda i, s: (s * 10 + i, 0)),
            pl.BlockSpec((B, 16), lambda i, s: (s * 10 + i, 0)),
            pl.BlockSpec((B, 144), lambda i, s: (s * 10 + i, 0)),
        ],
        out_shape=[
            jax.ShapeDtypeStruct((2 * NN, 128), jnp.float32),
            jax.ShapeDtypeStruct((2 * NN, 16), jnp.float32),
            jax.ShapeDtypeStruct((2 * NN, 144), jnp.float32),
        ],
    )(x, W1, Bsrc1, Bdst1)


def _e1_body(h1s, a1s, pack, init1, out,
             acc, ib, hbuf, asb, adb, msg, sem):
    c = lax.axis_index("c")
    s = lax.axis_index("s")
    rows = NACC // NS
    pltpu.sync_copy(init1.at[pl.ds(NACC * c + rows * s, rows)],
                    acc.at[pl.ds(rows * s, rows)])
    plsc.subcore_barrier()

    @pl.loop(0, NCH1)
    def _chunk(ci):
        pltpu.sync_copy(pack.at[c, s, ci], ib)
        pltpu.async_copy(h1s.at[ib.at[0]], hbuf, sem).wait()
        pltpu.async_copy(a1s.at[ib.at[0]], asb, sem).wait()
        pltpu.async_copy(a1s.at[ib.at[1]], adb, sem).wait()

        @pl.loop(0, K)
        def _edge(e):
            iot = _iota16()
            perm4 = (iot & 3) + 4
            arow = asb[e, :]
            drow = _gat(adb[e, :], perm4)
            w = jnp.where(iot < 4, jnp.exp(_lrelu(arow + drow)), 0.0)
            for h in range(4):
                bc = _bcast(w, h)
                for cc in range(2):
                    k = h * 2 + cc
                    msg[e, pl.ds(k * 16, 16)] = bc * hbuf[e, pl.ds(k * 16, 16)]
            msg[e, pl.ds(128, 16)] = w

        pltpu.sync_copy(msg, acc.at[ib.at[2]], add=True)

    plsc.subcore_barrier()
    pltpu.sync_copy(acc.at[pl.ds(rows * s, rows)],
                    out.at[pl.ds(NACC * c + rows * s, rows)])


def _e1(h1s, a1s, pack, init1):
    return pl.kernel(
        _e1_body,
        out_type=jax.ShapeDtypeStruct((2 * NACC, 144), jnp.float32),
        compiler_params=pltpu.CompilerParams(use_tc_tiling_on_sc=False),
        mesh=plsc.VectorSubcoreMesh(**_MESH),
        scratch_types=[
            pltpu.VMEM_SHARED((NACC, 144), jnp.float32),
            pltpu.VMEM((3, K), jnp.int32),
            pltpu.VMEM((K, 128), jnp.float32),
            pltpu.VMEM((K, 16), jnp.float32),
            pltpu.VMEM((K, 16), jnp.float32),
            pltpu.VMEM((K, 144), jnp.float32),
            pltpu.SemaphoreType.DMA,
        ],
    )(h1s, a1s, pack, init1)


def _stage_c_body(numa_ref, numb_ref, b1_ref, w2_ref, bs_ref, bd_ref,
                  h2r_ref, a2_ref, d2i_ref):
    s = pl.program_id(1)
    parts = []
    for ref in (numa_ref, numb_ref):
        x = ref[...]
        for h in range(4):
            parts.append(x[:, 32 * h:32 * (h + 1)] / x[:, 128 + h:129 + h])
    h1r = jnp.maximum(jnp.concatenate(parts, axis=1) + b1_ref[...], 0.0)
    h2h = jnp.dot(h1r, w2_ref[...], preferred_element_type=jnp.float32)
    h2r_ref[...] = h2h
    ps = jnp.dot(h2h, bs_ref[0], preferred_element_type=jnp.float32)
    pd = jnp.dot(h2h, bd_ref[0], preferred_element_type=jnp.float32)
    pa = jnp.concatenate([ps, pd], axis=1)

    @pl.when(s == 0)
    def _():
        a2_ref[...] = pa

    @pl.when(s == 1)
    def _():
        a2 = a2_ref[...] + pa
        a2_ref[...] = a2
        wself = jnp.exp(_lrelu(a2[:, :8] + a2[:, 8:]))
        d2i_ref[...] = jnp.concatenate([wself, wself], axis=1)


def _stage_c(num1a, num1b, b1r, W2perm, Bsrc2, Bdst2):
    B = 1000
    grid = (NN // B, 2)
    return pl.pallas_call(
        _stage_c_body,
        grid=grid,
        in_specs=[
            pl.BlockSpec((B, 144), lambda i, s: (i, 0)),
            pl.BlockSpec((B, 144), lambda i, s: (i, 0)),
            pl.BlockSpec((1, HID), lambda i, s: (0, 0)),
            pl.BlockSpec((HID, 1024), lambda i, s: (0, s)),
            pl.BlockSpec((1, 1024, 8), lambda i, s: (s, 0, 0)),
            pl.BlockSpec((1, 1024, 8), lambda i, s: (s, 0, 0)),
        ],
        out_specs=[
            pl.BlockSpec((B, 1024), lambda i, s: (s * 10 + i, 0)),
            pl.BlockSpec((B, 16), lambda i, s: (i, 0)),
            pl.BlockSpec((B, 16), lambda i, s: (i, 0)),
        ],
        out_shape=[
            jax.ShapeDtypeStruct((2 * NN, 1024), jnp.float32),
            jax.ShapeDtypeStruct((NN, 16), jnp.float32),
            jax.ShapeDtypeStruct((NN, 16), jnp.float32),
        ],
    )(num1a, num1b, b1r, W2perm, Bsrc2, Bdst2)


def _e2_body(a2, pack, d2i, out, acc, ib, asb, adb, msg, sem):
    c = lax.axis_index("c")
    s = lax.axis_index("s")
    rows = NACC // NS
    pltpu.sync_copy(d2i.at[pl.ds(rows * s, rows)],
                    acc.at[pl.ds(rows * s, rows)])
    plsc.subcore_barrier()

    @pl.loop(0, NCH1)
    def _chunk(ci):
        pltpu.sync_copy(pack.at[s, ci], ib)
        pltpu.async_copy(a2.at[ib.at[0]], asb, sem).wait()
        pltpu.async_copy(a2.at[ib.at[1]], adb, sem).wait()

        @pl.loop(0, K)
        def _edge(e):
            iot = _iota16()
            perm8 = (iot & 7) + 8
            arow = asb[e, :]
            drow = _gat(adb[e, :], perm8)
            msg[e, :] = jnp.where(iot < 8, jnp.exp(_lrelu(arow + drow)), 0.0)

        pltpu.sync_copy(msg, acc.at[ib.at[1]], add=True)

    plsc.subcore_barrier()

    @pl.when(c == 0)
    def _():
        pltpu.sync_copy(acc.at[pl.ds(rows * s, rows)],
                        out.at[pl.ds(rows * s, rows)])


def _e2(a2p, pack, d2ip):
    return pl.kernel(
        _e2_body,
        out_type=jax.ShapeDtypeStruct((NACC, 16), jnp.float32),
        compiler_params=pltpu.CompilerParams(use_tc_tiling_on_sc=False),
        mesh=plsc.VectorSubcoreMesh(**_MESH),
        scratch_types=[
            pltpu.VMEM_SHARED((NACC, 16), jnp.float32),
            pltpu.VMEM((2, K), jnp.int32),
            pltpu.VMEM((K, 16), jnp.float32),
            pltpu.VMEM((K, 16), jnp.float32),
            pltpu.VMEM((K, 16), jnp.float32),
            pltpu.SemaphoreType.DMA,
        ],
    )(a2p, pack, d2ip)


def _e3_body(h2r, a2, den2, pack, zeros, out,
             acc, ib, hbuf, asb, adb, dnb, msg, sem):
    c = lax.axis_index("c")
    s = lax.axis_index("s")
    rows = NACC // NS
    pltpu.sync_copy(zeros.at[pl.ds(rows * s, rows)],
                    acc.at[pl.ds(rows * s, rows)])
    plsc.subcore_barrier()

    @pl.loop(0, NCH3)
    def _chunk(ci):
        pltpu.sync_copy(pack.at[c, s, ci], ib)
        pltpu.async_copy(h2r.at[ib.at[0]], hbuf, sem).wait()
        pltpu.async_copy(a2.at[ib.at[1]], asb, sem).wait()
        pltpu.async_copy(a2.at[ib.at[2]], adb, sem).wait()
        pltpu.async_copy(den2.at[ib.at[2]], dnb, sem).wait()

        @pl.loop(0, K)
        def _edge(e):
            iot = _iota16()
            perm8 = (iot & 7) + 8
            arow = asb[e, :]
            drow = _gat(adb[e, :], perm8)
            w = jnp.exp(_lrelu(arow + drow))
            coef = w * 0.125 / dnb[e, :]
            bcs = [_bcast(coef, h) for h in range(8)]
            for cc in range(8):
                m = bcs[0] * hbuf[e, pl.ds(cc * 16, 16)]
                for h in range(1, 8):
                    m = m + bcs[h] * hbuf[e, pl.ds((h * 8 + cc) * 16, 16)]
                msg[e, pl.ds(cc * 16, 16)] = m

        pltpu.sync_copy(msg, acc.at[ib.at[2]], add=True)

    plsc.subcore_barrier()
    pltpu.sync_copy(acc.at[pl.ds(rows * s, rows)],
                    out.at[pl.ds(NACC * c + rows * s, rows)])


def _e3(h2r, a2p, den2, pack, zeros):
    return pl.kernel(
        _e3_body,
        out_type=jax.ShapeDtypeStruct((2 * NACC, 128), jnp.float32),
        compiler_params=pltpu.CompilerParams(use_tc_tiling_on_sc=False),
        mesh=plsc.VectorSubcoreMesh(**_MESH),
        scratch_types=[
            pltpu.VMEM_SHARED((NACC, 128), jnp.float32),
            pltpu.VMEM((3, K), jnp.int32),
            pltpu.VMEM((K, 1024), jnp.float32),
            pltpu.VMEM((K, 16), jnp.float32),
            pltpu.VMEM((K, 16), jnp.float32),
            pltpu.VMEM((K, 16), jnp.float32),
            pltpu.VMEM((K, 128), jnp.float32),
            pltpu.SemaphoreType.DMA,
        ],
    )(h2r, a2p, den2, pack, zeros)


def _stage_f_body(na_ref, nb_ref, b2_ref, batch_ref, wc_ref, bc_ref, out_ref,
                  sums, cnt):
    i = pl.program_id(0)
    h2 = jnp.concatenate([na_ref[...], nb_ref[...]], axis=1) + b2_ref[...]
    bv = batch_ref[0, 0, :]
    onehot = (bv[:, None] == lax.broadcasted_iota(jnp.int32, (1, NUM_GRAPHS), 1)
              ).astype(jnp.float32)
    psum = lax.dot_general(onehot, h2, (((0,), (0,)), ((), ())),
                           preferred_element_type=jnp.float32)
    ones = jnp.ones((onehot.shape[0], 128), jnp.float32)
    pcnt = lax.dot_general(onehot, ones, (((0,), (0,)), ((), ())),
                           preferred_element_type=jnp.float32)

    @pl.when(i == 0)
    def _():
        sums[...] = jnp.zeros_like(sums)
        cnt[...] = jnp.zeros_like(cnt)

    sums[...] += psum
    cnt[...] += pcnt

    @pl.when(i == pl.num_programs(0) - 1)
    def _():
        pooled = sums[...] / jnp.maximum(cnt[...][:, :1], 1.0)
        out_ref[...] = jnp.dot(pooled, wc_ref[...],
                               preferred_element_type=jnp.float32) + bc_ref[...]


def _stage_f(num2a, num2b, b2r, batch3, Wc, bcr):
    B = 1000
    return pl.pallas_call(
        _stage_f_body,
        grid=(NN // B,),
        in_specs=[
            pl.BlockSpec((B, 128), lambda i: (i, 0)),
            pl.BlockSpec((B, 128), lambda i: (i, 0)),
            pl.BlockSpec((1, HID), lambda i: (0, 0)),
            pl.BlockSpec((1, 1, B), lambda i: (i, 0, 0)),
            pl.BlockSpec((HID, NUM_CLASSES), lambda i: (0, 0)),
            pl.BlockSpec((1, NUM_CLASSES), lambda i: (0, 0)),
        ],
        out_specs=pl.BlockSpec((NUM_GRAPHS, NUM_CLASSES), lambda i: (0, 0)),
        out_shape=jax.ShapeDtypeStruct((NUM_GRAPHS, NUM_CLASSES), jnp.float32),
        scratch_shapes=[
            pltpu.VMEM((NUM_GRAPHS, HID), jnp.float32),
            pltpu.VMEM((NUM_GRAPHS, 128), jnp.float32),
        ],
    )(num2a, num2b, b2r, batch3, Wc, bcr)


def kernel(x, edge_index, batch, W1, a_src1, a_dst1, b1, W2, a_src2, a_dst2,
           b2, Wc, bc):
    f32 = jnp.float32
    src = edge_index[0]
    dst = edge_index[1]

    e1pack = jnp.stack([
        jnp.stack([src, dst, dst]),
        jnp.stack([src + NN, dst + NN, dst]),
    ]).reshape(2, 3, NS, NCH1, K).transpose(0, 2, 3, 1, 4)
    e2pack = jnp.stack([src, dst]).reshape(2, NS, NCH1, K) \
        .transpose(1, 2, 0, 3)
    loop = jnp.arange(NN, dtype=src.dtype)
    src3 = jnp.concatenate([src, loop, jnp.zeros((NPAD,), src.dtype)])
    dst3 = jnp.concatenate([dst, loop, jnp.full((NPAD,), NN, src.dtype)])
    e3pack = jnp.stack([
        jnp.stack([src3, src3, dst3]),
        jnp.stack([src3 + NN, src3, dst3]),
    ]).reshape(2, 3, NS, NCH3, K).transpose(0, 2, 3, 1, 4)

    eye8 = jnp.eye(HEADS, dtype=f32)
    B1s = (a_src1[:, :, None] * eye8[:, None, :]).reshape(HID, HEADS)
    B1d = (a_dst1[:, :, None] * eye8[:, None, :]).reshape(HID, HEADS)
    Bsrc1 = jnp.stack([B1s[0:128, 0:4], B1s[128:256, 4:8]])
    Bdst1 = jnp.stack([B1d[0:128, 0:4], B1d[128:256, 4:8]])
    a2s = a_src2.reshape(HEADS, 2, 128)
    a2d = a_dst2.reshape(HEADS, 2, 128)
    Bsrc2 = jnp.stack(
        [(a2s[:, s, :, None] * eye8[:, None, :]).reshape(1024, HEADS)
         for s in range(2)])
    Bdst2 = jnp.stack(
        [(a2d[:, s, :, None] * eye8[:, None, :]).reshape(1024, HEADS)
         for s in range(2)])
    W2perm = W2.reshape(HID, HEADS, 2, 128).transpose(0, 2, 1, 3) \
        .reshape(HID, 2 * 1024)

    b1r = b1.reshape(1, HID)
    b2r = b2.reshape(1, HID)
    bcr = bc.reshape(1, NUM_CLASSES)
    batch3 = batch.reshape(NN // 1000, 1, 1000)
    z0 = jnp.zeros((NACC, 128), f32)

    h1s, a1s, init1 = _stage_a(x, W1, Bsrc1, Bdst1)
    zpad = jnp.zeros((NACC - NN, 144), f32)
    init1p = jnp.concatenate([init1[:NN], zpad, init1[NN:], zpad])
    num1 = _e1(h1s, a1s, e1pack, init1p)
    h2r, a2t, d2i = _stage_c(num1[:NN], num1[NACC:NACC + NN], b1r, W2perm,
                             Bsrc2, Bdst2)
    a2p = jnp.concatenate([a2t, jnp.zeros((NACC - NN, 16), f32)])
    d2ip = jnp.concatenate([d2i, jnp.ones((NACC - NN, 16), f32)])
    den2 = _e2(a2p, e2pack, d2ip)
    num2 = _e3(h2r, a2p, den2, e3pack, z0)
    return _stage_f(num2[:NN], num2[NACC:NACC + NN], b2r, batch3, Wc, bcr)
